# baseline (speedup 1.0000x reference)
import jax
import jax.numpy as jnp
from jax import lax
from jax.experimental import pallas as pl
from jax.experimental.pallas import tpu as pltpu
from jax.experimental.pallas import tpu_sc as plsc

D_MODEL = 64
NC, NS = 2, 16
NW = NC * NS
CHUNK = 128


def _gather_body(tok_hbm, tab_hbm, out_hbm, idx_v, rows_v, sem):
    b_per_w = idx_v.shape[0]
    nchunks = b_per_w // CHUNK
    wid = lax.axis_index("c") * NS + lax.axis_index("s")
    base = wid * b_per_w
    pltpu.sync_copy(tok_hbm.at[pl.ds(base, b_per_w)], idx_v)

    def chunk(i, carry):
        off = i * CHUNK
        pltpu.async_copy(
            tab_hbm.at[idx_v.at[pl.ds(off, CHUNK)]], rows_v, sem
        ).wait()
        pltpu.sync_copy(rows_v, out_hbm.at[pl.ds(base + off, CHUNK)])
        return carry

    lax.fori_loop(0, nchunks, chunk, 0)


def kernel(token_ids, embedding_table):
    batch, ctx = token_ids.shape
    n = batch * ctx
    b_per_w = n // NW
    flat = token_ids.reshape(n).astype(jnp.int32)

    mesh = plsc.VectorSubcoreMesh(core_axis_name="c", subcore_axis_name="s")
    run = pl.kernel(
        _gather_body,
        out_type=jax.ShapeDtypeStruct((n, D_MODEL), jnp.float32),
        mesh=mesh,
        scratch_types=[
            pltpu.VMEM((b_per_w,), jnp.int32),
            pltpu.VMEM((CHUNK, D_MODEL), jnp.float32),
            pltpu.SemaphoreType.DMA,
        ],
        compiler_params=pltpu.CompilerParams(use_tc_tiling_on_sc=False),
    )
    out = run(flat, embedding_table)
    return out.reshape(batch, ctx, D_MODEL)


# speedup vs baseline: 1.0400x; 1.0400x over previous
import jax
import jax.numpy as jnp
from jax import lax
from jax.experimental import pallas as pl
from jax.experimental.pallas import tpu as pltpu
from jax.experimental.pallas import tpu_sc as plsc

D_MODEL = 64
NC, NS = 2, 16
NW = NC * NS
CHUNK = 640
NBUF = 2


def _gather_body(tok_hbm, tab_hbm, out_hbm, idx_v, rows_v, g0, g1, w0, w1):
    b_per_w = idx_v.shape[0]
    nchunks = b_per_w // CHUNK
    nsteps = nchunks // NBUF
    gsem = (g0, g1)
    wsem = (w0, w1)
    wid = lax.axis_index("c") * NS + lax.axis_index("s")
    base = wid * b_per_w
    pltpu.sync_copy(tok_hbm.at[pl.ds(base, b_per_w)], idx_v)

    def start_gather(i, b):
        pltpu.async_copy(
            tab_hbm.at[idx_v.at[pl.ds(i * CHUNK, CHUNK)]], rows_v.at[b], gsem[b]
        )

    def wait_gather(b):
        pltpu.make_async_copy(
            tab_hbm.at[idx_v.at[pl.ds(0, CHUNK)]], rows_v.at[b], gsem[b]
        ).wait()

    def start_writeback(i, b):
        pltpu.async_copy(
            rows_v.at[b], out_hbm.at[pl.ds(base + i * CHUNK, CHUNK)], wsem[b]
        )

    def wait_writeback(b):
        pltpu.make_async_copy(
            rows_v.at[b], out_hbm.at[pl.ds(base, CHUNK)], wsem[b]
        ).wait()

    for b in range(NBUF):
        start_gather(b, b)

    def step(s, carry):
        for b in range(NBUF):
            wait_gather(b)
            start_writeback(s * NBUF + b, b)
        for b in range(NBUF):
            wait_writeback(b)
            start_gather((s + 1) * NBUF + b, b)
        return carry

    lax.fori_loop(0, nsteps - 1, step, 0)

    last = (nsteps - 1) * NBUF
    for b in range(NBUF):
        wait_gather(b)
        start_writeback(last + b, b)
    for b in range(NBUF):
        wait_writeback(b)


def kernel(token_ids, embedding_table):
    batch, ctx = token_ids.shape
    n = batch * ctx
    b_per_w = n // NW
    flat = token_ids.reshape(n).astype(jnp.int32)

    mesh = plsc.VectorSubcoreMesh(core_axis_name="c", subcore_axis_name="s")
    run = pl.kernel(
        _gather_body,
        out_type=jax.ShapeDtypeStruct((n, D_MODEL), jnp.float32),
        mesh=mesh,
        scratch_types=[
            pltpu.VMEM((b_per_w,), jnp.int32),
            pltpu.VMEM((NBUF, CHUNK, D_MODEL), jnp.float32),
            pltpu.SemaphoreType.DMA,
            pltpu.SemaphoreType.DMA,
            pltpu.SemaphoreType.DMA,
            pltpu.SemaphoreType.DMA,
        ],
        compiler_params=pltpu.CompilerParams(use_tc_tiling_on_sc=False),
    )
    out = run(flat, embedding_table)
    return out.reshape(batch, ctx, D_MODEL)


# speedup vs baseline: 1.0410x; 1.0010x over previous
import jax
import jax.numpy as jnp
from jax import lax
from jax.experimental import pallas as pl
from jax.experimental.pallas import tpu as pltpu
from jax.experimental.pallas import tpu_sc as plsc

D_MODEL = 64
NC, NS = 2, 16
NW = NC * NS
C = 80
G = 8


def _gather_body(tok_hbm, tab_hbm, out_hbm, idx_v, bufs, sg0, sg1, sw0, sw1):
    b_per_w = idx_v.shape[0]
    nch = b_per_w // C
    nhw = nch // G
    sg = (sg0, sg1)
    sw = (sw0, sw1)
    wid = lax.axis_index("c") * NS + lax.axis_index("s")
    base = wid * b_per_w
    pltpu.sync_copy(tok_hbm.at[pl.ds(base, b_per_w)], idx_v)

    def fire_gather(h):
        g = h % 2
        for j in range(G):
            pltpu.async_copy(
                tab_hbm.at[idx_v.at[pl.ds((h * G + j) * C, C)]],
                bufs.at[g * G + j], sg[g],
            )

    def drain_gather(h):
        g = h % 2
        for j in range(G):
            pltpu.make_async_copy(
                tab_hbm.at[idx_v.at[pl.ds(0, C)]], bufs.at[g * G + j], sg[g]
            ).wait()

    def fire_wb(h):
        g = h % 2
        for j in range(G):
            pltpu.async_copy(
                bufs.at[g * G + j],
                out_hbm.at[pl.ds(base + (h * G + j) * C, C)], sw[g],
            )

    def drain_wb(g):
        for j in range(G):
            pltpu.make_async_copy(
                bufs.at[g * G + j], out_hbm.at[pl.ds(base, C)], sw[g]
            ).wait()

    fire_gather(0)
    for h in range(nhw):
        if h + 1 < nhw:
            if h >= 1:
                drain_wb((h + 1) % 2)
            fire_gather(h + 1)
        drain_gather(h)
        fire_wb(h)
    drain_wb(0)
    drain_wb(1)


def kernel(token_ids, embedding_table):
    batch, ctx = token_ids.shape
    n = batch * ctx
    b_per_w = n // NW
    flat = token_ids.reshape(n).astype(jnp.int32)

    mesh = plsc.VectorSubcoreMesh(core_axis_name="c", subcore_axis_name="s")
    run = pl.kernel(
        _gather_body,
        out_type=jax.ShapeDtypeStruct((n, D_MODEL), jnp.float32),
        mesh=mesh,
        scratch_types=[
            pltpu.VMEM((b_per_w,), jnp.int32),
            pltpu.VMEM((2 * G, C, D_MODEL), jnp.float32),
            pltpu.SemaphoreType.DMA,
            pltpu.SemaphoreType.DMA,
            pltpu.SemaphoreType.DMA,
            pltpu.SemaphoreType.DMA,
        ],
        compiler_params=pltpu.CompilerParams(use_tc_tiling_on_sc=False),
    )
    out = run(flat, embedding_table)
    return out.reshape(batch, ctx, D_MODEL)
